# baseline (speedup 1.0000x reference)
import functools

import jax
import jax.numpy as jnp
from jax import lax
from jax.experimental import pallas as pl
from jax.experimental.pallas import tpu as pltpu
from jax.experimental.pallas import tpu_sc as plsc

DIM = 1024
CUTS = (0, 2000, 8000, 16000, 32000)
VTOT = 32000
VT = 2000
TT = 256
N_TOK = 4096
S = N_TOK + 4 * TT
KS = tuple((CUTS[i + 1] - CUTS[i]) // VT for i in range(4))
TIMAX = N_TOK // TT
MAX_ITEMS = 144
NEG = -1e30

_NW = 32


def _sc_build(slot, tf):
    mesh = plsc.VectorSubcoreMesh(core_axis_name="c", subcore_axis_name="s",
                                  num_cores=1)
    nw = 16
    icnk = S // nw
    scnk = N_TOK // nw

    @functools.partial(
        pl.kernel,
        mesh=mesh,
        out_type=(jax.ShapeDtypeStruct((S,), jnp.int32),
                  jax.ShapeDtypeStruct((S,), jnp.int32)),
        scratch_types=[
            pltpu.VMEM((icnk,), jnp.int32),
            pltpu.VMEM((scnk,), jnp.int32),
            pltpu.VMEM((scnk,), jnp.int32),
            pltpu.VMEM((scnk,), jnp.int32),
            pltpu.SemaphoreType.DMA,
        ],
    )
    def k(slot_hbm, tf_hbm, gi_hbm, ts_hbm, ibuf, idx_v, tok_v, tv_v, sem):
        wid = lax.axis_index("s")
        for i in range(icnk // 16):
            ibuf[pl.ds(i * 16, 16)] = jnp.zeros((16,), jnp.int32)
        pltpu.sync_copy(ibuf, gi_hbm.at[pl.ds(wid * icnk, icnk)])
        for i in range(icnk // 16):
            ibuf[pl.ds(i * 16, 16)] = jnp.full((16,), -1, jnp.int32)
        pltpu.sync_copy(ibuf, ts_hbm.at[pl.ds(wid * icnk, icnk)])
        plsc.subcore_barrier()
        base = wid * scnk
        pltpu.sync_copy(slot_hbm.at[pl.ds(base, scnk)], idx_v)
        pltpu.sync_copy(tf_hbm.at[pl.ds(base, scnk)], tv_v)
        for i in range(scnk // 16):
            tok_v[pl.ds(i * 16, 16)] = (
                lax.iota(jnp.int32, 16) + (base + i * 16))
        pltpu.async_copy(tok_v, gi_hbm.at[idx_v], sem).wait()
        pltpu.async_copy(tv_v, ts_hbm.at[idx_v], sem).wait()

    return k(slot, tf)


def _sc_gather_rows(xf, gather_idx):
    mesh = plsc.VectorSubcoreMesh(core_axis_name="c", subcore_axis_name="s")
    nchunk = 4
    chunk = S // _NW // nchunk

    @functools.partial(
        pl.kernel,
        mesh=mesh,
        out_type=jax.ShapeDtypeStruct((S, DIM), jnp.float32),
        scratch_types=[
            pltpu.VMEM((nchunk, chunk), jnp.int32),
            pltpu.VMEM((chunk, DIM), jnp.float32),
            pltpu.VMEM((chunk, DIM), jnp.float32),
            pltpu.SemaphoreType.DMA,
            pltpu.SemaphoreType.DMA,
        ],
    )
    def k(x_hbm, idx_hbm, out_hbm, idx_v, rows_a, rows_b, sem_a, sem_b):
        wid = lax.axis_index("s") * 2 + lax.axis_index("c")
        w0 = wid * (nchunk * chunk)
        for j in range(nchunk):
            pltpu.sync_copy(idx_hbm.at[pl.ds(w0 + j * chunk, chunk)],
                            idx_v.at[j])
        bufs = (rows_a, rows_b)
        sems = (sem_a, sem_b)
        cps = []
        for j in range(nchunk):
            cps.append(pltpu.async_copy(
                x_hbm.at[idx_v.at[j]], bufs[j % 2], sems[j % 2]))
            if j >= 1:
                cps[j - 1].wait()
                pltpu.sync_copy(bufs[(j - 1) % 2],
                                out_hbm.at[pl.ds(w0 + (j - 1) * chunk, chunk)])
        cps[nchunk - 1].wait()
        pltpu.sync_copy(bufs[(nchunk - 1) % 2],
                        out_hbm.at[pl.ds(w0 + (nchunk - 1) * chunk, chunk)])

    return k(xf, gather_idx)


def _sc_unsort(nll_s, cls_s, slot):
    mesh = plsc.VectorSubcoreMesh(core_axis_name="c", subcore_axis_name="s")
    chunk = N_TOK // _NW

    @functools.partial(
        pl.kernel,
        mesh=mesh,
        out_type=(jax.ShapeDtypeStruct((N_TOK,), jnp.float32),
                  jax.ShapeDtypeStruct((N_TOK,), jnp.int32)),
        scratch_types=[
            pltpu.VMEM((chunk,), jnp.int32),
            pltpu.VMEM((chunk,), jnp.float32),
            pltpu.VMEM((chunk,), jnp.int32),
            pltpu.SemaphoreType.DMA,
            pltpu.SemaphoreType.DMA,
        ],
    )
    def k(nll_hbm, cls_hbm, slot_hbm, out_nll, out_cls,
          idx_v, val_f, val_i, sem, sem2):
        wid = lax.axis_index("s") * 2 + lax.axis_index("c")
        base = wid * chunk
        pltpu.sync_copy(slot_hbm.at[pl.ds(base, chunk)], idx_v)
        cp = pltpu.async_copy(nll_hbm.at[idx_v], val_f, sem)
        cp2 = pltpu.async_copy(cls_hbm.at[idx_v], val_i, sem2)
        cp.wait()
        pltpu.sync_copy(val_f, out_nll.at[pl.ds(base, chunk)])
        cp2.wait()
        pltpu.sync_copy(val_i, out_cls.at[pl.ds(base, chunk)])

    return k(nll_s, cls_s, slot)


def _body(tt_ref, wb_ref, fi_ref, la_ref, sk_ref,
          x_ref, tgt_ref, rw_ref, w_ref,
          nll_ref, cls_ref, loss_ref, used_ref,
          m_ref, s_ref, ami_ref, tgl_ref, rnll_ref,
          lacc_ref, uacc_ref):
    g = pl.program_id(0)

    @pl.when(g == 0)
    def _zero():
        lacc_ref[0] = 0.0
        uacc_ref[0] = 0

    @pl.when(sk_ref[g] == 0)
    def _work():
        tile = tt_ref[g]
        rows = pl.ds(tile * TT, TT)
        base = wb_ref[g] * VT
        tvals = tgt_ref[...]
        xt = x_ref[...]
        c_tok = ((tvals >= CUTS[1]).astype(jnp.int32)
                 + (tvals >= CUTS[2]).astype(jnp.int32)
                 + (tvals >= CUTS[3]).astype(jnp.int32))

        @pl.when(fi_ref[g] == 1)
        def _init():
            rl = jax.lax.dot_general(xt, rw_ref[...], (((1,), (1,)), ((), ())),
                                     preferred_element_type=jnp.float32)
            col8 = jax.lax.broadcasted_iota(jnp.int32, (TT, 8), 1)
            rl = jnp.where(col8 < 4, rl, NEG)
            rmax = jnp.max(rl, axis=1, keepdims=True)
            rlse = rmax + jnp.log(jnp.sum(jnp.exp(rl - rmax), axis=1,
                                          keepdims=True))
            rlog_c = jnp.sum(jnp.where(col8 == c_tok, rl, 0.0), axis=1,
                             keepdims=True)
            rnll_ref[rows, :] = rlse - rlog_c
            m_ref[rows, :] = jnp.full((TT, 1), NEG, jnp.float32)
            s_ref[rows, :] = jnp.zeros((TT, 1), jnp.float32)
            ami_ref[rows, :] = jnp.zeros((TT, 1), jnp.int32)
            tgl_ref[rows, :] = jnp.zeros((TT, 1), jnp.float32)

        c_tile = ((base >= CUTS[1]).astype(jnp.int32)
                  + (base >= CUTS[2]).astype(jnp.int32)
                  + (base >= CUTS[3]).astype(jnp.int32))
        mask = c_tok == c_tile

        logits = jax.lax.dot_general(xt, w_ref[...], (((1,), (1,)), ((), ())),
                                     preferred_element_type=jnp.float32)
        tile_max = jnp.max(logits, axis=1, keepdims=True)
        tile_arg = (jnp.argmax(logits, axis=1, keepdims=True).astype(jnp.int32)
                    + base)

        m_old = m_ref[rows, :]
        s_old = s_ref[rows, :]
        m_new = jnp.where(mask, jnp.maximum(m_old, tile_max), m_old)
        sumexp = jnp.sum(jnp.exp(logits - m_new), axis=1, keepdims=True)
        s_new = jnp.where(mask, s_old * jnp.exp(m_old - m_new) + sumexp, s_old)
        m_ref[rows, :] = m_new
        s_ref[rows, :] = s_new

        upd = mask & (tile_max > m_old)
        ami_ref[rows, :] = jnp.where(upd, tile_arg, ami_ref[rows, :])

        col = tvals - base
        hit = mask & (col >= 0) & (col < VT)
        lane = jax.lax.broadcasted_iota(jnp.int32, (TT, VT), 1)
        tl = jnp.sum(jnp.where(lane == col, logits, 0.0), axis=1,
                     keepdims=True)
        tgl_ref[rows, :] = jnp.where(hit, tl, tgl_ref[rows, :])

        @pl.when(la_ref[g] == 1)
        def _final():
            valid = (tvals >= 0) & (tvals < VTOT)
            lse = m_new + jnp.log(jnp.maximum(s_new, 1e-37))
            nll = lse - tgl_ref[rows, :] + rnll_ref[rows, :]
            nll = jnp.where(valid, nll, 0.0)
            nll_ref[rows, :] = nll
            cls_ref[rows, :] = jnp.where(valid, ami_ref[rows, :], -100)
            lacc_ref[0] = lacc_ref[0] + jnp.sum(nll)
            uacc_ref[0] = uacc_ref[0] + jnp.sum(valid.astype(jnp.int32))

    @pl.when(g == MAX_ITEMS - 1)
    def _emit():
        u = uacc_ref[0]
        used_ref[...] = jnp.full((1, 1), u, jnp.int32)
        loss_ref[...] = jnp.full(
            (1, 1), lacc_ref[0] / jnp.maximum(u, 1).astype(jnp.float32),
            jnp.float32)


def _run_tc(xs, tgtc, rw8, W, tok_tile, w_blk, first, last, skip):
    out_shapes = (
        jax.ShapeDtypeStruct((S, 1), jnp.float32),
        jax.ShapeDtypeStruct((S, 1), jnp.int32),
        jax.ShapeDtypeStruct((1, 1), jnp.float32),
        jax.ShapeDtypeStruct((1, 1), jnp.int32),
    )
    full = lambda shape: pl.BlockSpec(
        shape, lambda g, *refs: tuple(0 for _ in shape))
    grid_spec = pltpu.PrefetchScalarGridSpec(
        num_scalar_prefetch=5,
        grid=(MAX_ITEMS,),
        in_specs=[
            pl.BlockSpec((TT, DIM), lambda g, tt, wb, fi, la, sk: (tt[g], 0)),
            pl.BlockSpec((TT, 1), lambda g, tt, wb, fi, la, sk: (tt[g], 0)),
            pl.BlockSpec((8, DIM), lambda g, tt, wb, fi, la, sk: (0, 0)),
            pl.BlockSpec((VT, DIM), lambda g, tt, wb, fi, la, sk: (wb[g], 0)),
        ],
        out_specs=(
            full((S, 1)),
            full((S, 1)),
            full((1, 1)),
            full((1, 1)),
        ),
        scratch_shapes=[
            pltpu.VMEM((S, 1), jnp.float32),
            pltpu.VMEM((S, 1), jnp.float32),
            pltpu.VMEM((S, 1), jnp.int32),
            pltpu.VMEM((S, 1), jnp.float32),
            pltpu.VMEM((S, 1), jnp.float32),
            pltpu.SMEM((1,), jnp.float32),
            pltpu.SMEM((1,), jnp.int32),
        ],
    )
    return pl.pallas_call(
        _body,
        grid_spec=grid_spec,
        out_shape=out_shapes,
    )(tok_tile, w_blk, first, last, skip, xs, tgtc, rw8, W)


def _route(tf):
    c = ((tf >= CUTS[1]).astype(jnp.int32)
         + (tf >= CUTS[2]).astype(jnp.int32)
         + (tf >= CUTS[3]).astype(jnp.int32))
    onehot = (c[:, None] == jnp.arange(4)[None, :]).astype(jnp.int32)
    ranks_incl = jnp.cumsum(onehot, axis=0)
    counts = ranks_incl[-1]
    rank = jnp.take_along_axis(ranks_incl, c[:, None], axis=1)[:, 0] - 1
    tiles = (counts + TT - 1) // TT
    pc = tiles * TT
    off_pad = jnp.concatenate([jnp.zeros((1,), jnp.int32),
                               jnp.cumsum(pc)[:-1].astype(jnp.int32)])
    slot = off_pad[c] + rank

    cl_l, kk_l, tt_l = [], [], []
    for i in range(4):
        for k in range(KS[i]):
            for t in range(TIMAX):
                cl_l.append(i); kk_l.append(k); tt_l.append(t)
    cl_s = jnp.asarray(cl_l, jnp.int32)
    kk_s = jnp.asarray(kk_l, jnp.int32)
    tt_s = jnp.asarray(tt_l, jnp.int32)
    valid = tt_s < tiles[cl_s]
    perm = jnp.argsort((~valid).astype(jnp.int32), stable=True)[:MAX_ITEMS]
    icl, ik, itt = cl_s[perm], kk_s[perm], tt_s[perm]
    ivalid = valid[perm]
    ks_arr = jnp.asarray(KS, jnp.int32)
    cut_arr = jnp.asarray(CUTS[:4], jnp.int32)
    tok_tile = jnp.where(ivalid, off_pad[icl] // TT + itt, 0)
    w_blk = jnp.where(ivalid, cut_arr[icl] // VT + ik, 0)
    first = (ivalid & (ik == 0)).astype(jnp.int32)
    last = (ivalid & (ik == ks_arr[icl] - 1)).astype(jnp.int32)
    skip = (~ivalid).astype(jnp.int32)
    return slot, tok_tile, w_blk, first, last, skip


def kernel(x, target, W, router_W):
    in_shape = target.shape
    xf = x.reshape(-1, DIM)
    tf = target.reshape(-1)
    rw8 = jnp.zeros((8, DIM), jnp.float32).at[:4].set(router_W)

    slot, tok_tile, w_blk, first, last, skip = _route(tf)
    gather_idx, tgt_slot = _sc_build(slot, tf)
    xs = _sc_gather_rows(xf, gather_idx)
    nll_s, cls_s, loss, used = _run_tc(
        xs, tgt_slot.reshape(S, 1), rw8, W, tok_tile, w_blk, first, last,
        skip)
    nll, cls = _sc_unsort(nll_s.reshape(S), cls_s.reshape(S), slot)
    return (loss.reshape(()), used.reshape(()),
            nll.reshape(in_shape), cls.reshape(in_shape))


# speedup vs baseline: 1.0856x; 1.0856x over previous
import functools

import jax
import jax.numpy as jnp
from jax import lax
from jax.experimental import pallas as pl
from jax.experimental.pallas import tpu as pltpu
from jax.experimental.pallas import tpu_sc as plsc

DIM = 1024
CUTS = (0, 2000, 8000, 16000, 32000)
VTOT = 32000
VT = 2000
TT = 256
N_TOK = 4096
S = N_TOK + 4 * TT
KS = tuple((CUTS[i + 1] - CUTS[i]) // VT for i in range(4))
TIMAX = N_TOK // TT
MAX_ITEMS = 144
NEG = -1e30

_NW = 32


def _sc_gather_rows(xf, gather_idx, nrows):
    mesh = plsc.VectorSubcoreMesh(core_axis_name="c", subcore_axis_name="s")
    nchunk = 4
    chunk = nrows // _NW // nchunk

    @functools.partial(
        pl.kernel,
        mesh=mesh,
        out_type=jax.ShapeDtypeStruct((nrows, DIM), jnp.float32),
        scratch_types=[
            pltpu.VMEM((nchunk, chunk), jnp.int32),
            pltpu.VMEM((chunk, DIM), jnp.float32),
            pltpu.VMEM((chunk, DIM), jnp.float32),
            pltpu.SemaphoreType.DMA,
            pltpu.SemaphoreType.DMA,
            pltpu.SemaphoreType.DMA,
            pltpu.SemaphoreType.DMA,
        ],
    )
    def k(x_hbm, idx_hbm, out_hbm, idx_v, rows_a, rows_b,
          sem_a, sem_b, wsem_a, wsem_b):
        wid = lax.axis_index("s") * 2 + lax.axis_index("c")
        w0 = wid * (nchunk * chunk)
        for j in range(nchunk):
            pltpu.sync_copy(idx_hbm.at[pl.ds(w0 + j * chunk, chunk)],
                            idx_v.at[j])
        bufs = (rows_a, rows_b)
        sems = (sem_a, sem_b)
        wsems = (wsem_a, wsem_b)
        cps = [None] * nchunk
        wbs = [None] * nchunk
        cps[0] = pltpu.async_copy(x_hbm.at[idx_v.at[0]], bufs[0], sems[0])
        cps[1] = pltpu.async_copy(x_hbm.at[idx_v.at[1]], bufs[1], sems[1])
        for j in range(nchunk):
            cps[j].wait()
            wbs[j] = pltpu.async_copy(
                bufs[j % 2], out_hbm.at[pl.ds(w0 + j * chunk, chunk)],
                wsems[j % 2])
            nj = j + 2
            if nj < nchunk:
                wbs[j].wait()
                cps[nj] = pltpu.async_copy(
                    x_hbm.at[idx_v.at[nj]], bufs[nj % 2], sems[nj % 2])
        wbs[nchunk - 2].wait()
        wbs[nchunk - 1].wait()

    return k(xf, gather_idx)


def _sc_unsort(nll_s, cls_s, slot):
    mesh = plsc.VectorSubcoreMesh(core_axis_name="c", subcore_axis_name="s")
    chunk = N_TOK // _NW

    @functools.partial(
        pl.kernel,
        mesh=mesh,
        out_type=(jax.ShapeDtypeStruct((N_TOK,), jnp.float32),
                  jax.ShapeDtypeStruct((N_TOK,), jnp.int32)),
        scratch_types=[
            pltpu.VMEM((chunk,), jnp.int32),
            pltpu.VMEM((chunk,), jnp.float32),
            pltpu.VMEM((chunk,), jnp.int32),
            pltpu.SemaphoreType.DMA,
            pltpu.SemaphoreType.DMA,
        ],
    )
    def k(nll_hbm, cls_hbm, slot_hbm, out_nll, out_cls,
          idx_v, val_f, val_i, sem, sem2):
        wid = lax.axis_index("s") * 2 + lax.axis_index("c")
        base = wid * chunk
        pltpu.sync_copy(slot_hbm.at[pl.ds(base, chunk)], idx_v)
        cp = pltpu.async_copy(nll_hbm.at[idx_v], val_f, sem)
        cp2 = pltpu.async_copy(cls_hbm.at[idx_v], val_i, sem2)
        cp.wait()
        pltpu.sync_copy(val_f, out_nll.at[pl.ds(base, chunk)])
        cp2.wait()
        pltpu.sync_copy(val_i, out_cls.at[pl.ds(base, chunk)])

    return k(nll_s, cls_s, slot)


def _body(tt_ref, wb_ref, fi_ref, la_ref, sk_ref,
          x_ref, tgt_ref, rw_ref, w_ref,
          nll_ref, cls_ref, loss_ref, used_ref,
          m_ref, s_ref, ami_ref, tgl_ref, rnll_ref,
          lacc_ref, uacc_ref):
    g = pl.program_id(0)

    @pl.when(g == 0)
    def _zero():
        lacc_ref[0] = 0.0
        uacc_ref[0] = 0

    @pl.when(sk_ref[g] == 0)
    def _work():
        tile = tt_ref[g]
        rows = pl.ds(tile * TT, TT)
        base = wb_ref[g] * VT
        tvals = tgt_ref[...]
        xt = x_ref[...]
        c_tok = ((tvals >= CUTS[1]).astype(jnp.int32)
                 + (tvals >= CUTS[2]).astype(jnp.int32)
                 + (tvals >= CUTS[3]).astype(jnp.int32))

        @pl.when(fi_ref[g] == 1)
        def _init():
            rl = jax.lax.dot_general(xt, rw_ref[...], (((1,), (1,)), ((), ())),
                                     preferred_element_type=jnp.float32)
            col8 = jax.lax.broadcasted_iota(jnp.int32, (TT, 8), 1)
            rl = jnp.where(col8 < 4, rl, NEG)
            rmax = jnp.max(rl, axis=1, keepdims=True)
            rlse = rmax + jnp.log(jnp.sum(jnp.exp(rl - rmax), axis=1,
                                          keepdims=True))
            rlog_c = jnp.sum(jnp.where(col8 == c_tok, rl, 0.0), axis=1,
                             keepdims=True)
            rnll_ref[rows, :] = rlse - rlog_c
            m_ref[rows, :] = jnp.full((TT, 1), NEG, jnp.float32)
            s_ref[rows, :] = jnp.zeros((TT, 1), jnp.float32)
            ami_ref[rows, :] = jnp.zeros((TT, 1), jnp.int32)
            tgl_ref[rows, :] = jnp.zeros((TT, 1), jnp.float32)

        c_tile = ((base >= CUTS[1]).astype(jnp.int32)
                  + (base >= CUTS[2]).astype(jnp.int32)
                  + (base >= CUTS[3]).astype(jnp.int32))
        mask = c_tok == c_tile

        logits = jax.lax.dot_general(xt, w_ref[...], (((1,), (1,)), ((), ())),
                                     preferred_element_type=jnp.float32)
        tile_max = jnp.max(logits, axis=1, keepdims=True)
        tile_arg = (jnp.argmax(logits, axis=1, keepdims=True).astype(jnp.int32)
                    + base)

        m_old = m_ref[rows, :]
        s_old = s_ref[rows, :]
        m_new = jnp.where(mask, jnp.maximum(m_old, tile_max), m_old)
        sumexp = jnp.sum(jnp.exp(logits - m_new), axis=1, keepdims=True)
        s_new = jnp.where(mask, s_old * jnp.exp(m_old - m_new) + sumexp, s_old)
        m_ref[rows, :] = m_new
        s_ref[rows, :] = s_new

        upd = mask & (tile_max > m_old)
        ami_ref[rows, :] = jnp.where(upd, tile_arg, ami_ref[rows, :])

        col = tvals - base
        hit = mask & (col >= 0) & (col < VT)
        lane = jax.lax.broadcasted_iota(jnp.int32, (TT, VT), 1)
        tl = jnp.sum(jnp.where(lane == col, logits, 0.0), axis=1,
                     keepdims=True)
        tgl_ref[rows, :] = jnp.where(hit, tl, tgl_ref[rows, :])

        @pl.when(la_ref[g] == 1)
        def _final():
            valid = (tvals >= 0) & (tvals < VTOT)
            lse = m_new + jnp.log(jnp.maximum(s_new, 1e-37))
            nll = lse - tgl_ref[rows, :] + rnll_ref[rows, :]
            nll = jnp.where(valid, nll, 0.0)
            nll_ref[rows, :] = nll
            cls_ref[rows, :] = jnp.where(valid, ami_ref[rows, :], -100)
            lacc_ref[0] = lacc_ref[0] + jnp.sum(nll)
            uacc_ref[0] = uacc_ref[0] + jnp.sum(valid.astype(jnp.int32))

    @pl.when(g == MAX_ITEMS - 1)
    def _emit():
        u = uacc_ref[0]
        used_ref[...] = jnp.full((1, 1), u, jnp.int32)
        loss_ref[...] = jnp.full(
            (1, 1), lacc_ref[0] / jnp.maximum(u, 1).astype(jnp.float32),
            jnp.float32)


def _run_tc(xs, tgtc, rw8, W, tok_tile, w_blk, first, last, skip):
    out_shapes = (
        jax.ShapeDtypeStruct((S, 1), jnp.float32),
        jax.ShapeDtypeStruct((S, 1), jnp.int32),
        jax.ShapeDtypeStruct((1, 1), jnp.float32),
        jax.ShapeDtypeStruct((1, 1), jnp.int32),
    )
    full = lambda shape: pl.BlockSpec(
        shape, lambda g, *refs: tuple(0 for _ in shape))
    grid_spec = pltpu.PrefetchScalarGridSpec(
        num_scalar_prefetch=5,
        grid=(MAX_ITEMS,),
        in_specs=[
            pl.BlockSpec((TT, DIM), lambda g, tt, wb, fi, la, sk: (tt[g], 0)),
            pl.BlockSpec((TT, 1), lambda g, tt, wb, fi, la, sk: (tt[g], 0)),
            pl.BlockSpec((8, DIM), lambda g, tt, wb, fi, la, sk: (0, 0)),
            pl.BlockSpec((VT, DIM), lambda g, tt, wb, fi, la, sk: (wb[g], 0)),
        ],
        out_specs=(
            full((S, 1)),
            full((S, 1)),
            full((1, 1)),
            full((1, 1)),
        ),
        scratch_shapes=[
            pltpu.VMEM((S, 1), jnp.float32),
            pltpu.VMEM((S, 1), jnp.float32),
            pltpu.VMEM((S, 1), jnp.int32),
            pltpu.VMEM((S, 1), jnp.float32),
            pltpu.VMEM((S, 1), jnp.float32),
            pltpu.SMEM((1,), jnp.float32),
            pltpu.SMEM((1,), jnp.int32),
        ],
    )
    return pl.pallas_call(
        _body,
        grid_spec=grid_spec,
        out_shape=out_shapes,
    )(tok_tile, w_blk, first, last, skip, xs, tgtc, rw8, W)


def _route(tf):
    c = ((tf >= CUTS[1]).astype(jnp.int32)
         + (tf >= CUTS[2]).astype(jnp.int32)
         + (tf >= CUTS[3]).astype(jnp.int32))
    onehot = (c[:, None] == jnp.arange(4)[None, :]).astype(jnp.int32)
    ranks_incl = jnp.cumsum(onehot, axis=0)
    counts = ranks_incl[-1]
    rank = jnp.take_along_axis(ranks_incl, c[:, None], axis=1)[:, 0] - 1
    tiles = (counts + TT - 1) // TT
    pc = tiles * TT
    off_pad = jnp.concatenate([jnp.zeros((1,), jnp.int32),
                               jnp.cumsum(pc)[:-1].astype(jnp.int32)])
    slot = off_pad[c] + rank

    cl_l, kk_l, tt_l = [], [], []
    for i in range(4):
        for k in range(KS[i]):
            for t in range(TIMAX):
                cl_l.append(i); kk_l.append(k); tt_l.append(t)
    cl_s = jnp.asarray(cl_l, jnp.int32)
    kk_s = jnp.asarray(kk_l, jnp.int32)
    tt_s = jnp.asarray(tt_l, jnp.int32)
    valid = tt_s < tiles[cl_s]
    perm = jnp.argsort((~valid).astype(jnp.int32), stable=True)[:MAX_ITEMS]
    icl, ik, itt = cl_s[perm], kk_s[perm], tt_s[perm]
    ivalid = valid[perm]
    ks_arr = jnp.asarray(KS, jnp.int32)
    cut_arr = jnp.asarray(CUTS[:4], jnp.int32)
    tok_tile = jnp.where(ivalid, off_pad[icl] // TT + itt, 0)
    w_blk = jnp.where(ivalid, cut_arr[icl] // VT + ik, 0)
    first = (ivalid & (ik == 0)).astype(jnp.int32)
    last = (ivalid & (ik == ks_arr[icl] - 1)).astype(jnp.int32)
    skip = (~ivalid).astype(jnp.int32)
    return slot, tok_tile, w_blk, first, last, skip


def kernel(x, target, W, router_W):
    in_shape = target.shape
    xf = x.reshape(-1, DIM)
    tf = target.reshape(-1)
    rw8 = jnp.zeros((8, DIM), jnp.float32).at[:4].set(router_W)

    slot, tok_tile, w_blk, first, last, skip = _route(tf)
    gather_idx = jnp.zeros((S,), jnp.int32).at[slot].set(
        jnp.arange(N_TOK, dtype=jnp.int32))
    tgt_slot = jnp.full((S,), -1, jnp.int32).at[slot].set(tf)
    xs = _sc_gather_rows(xf, gather_idx, S)
    nll_s, cls_s, loss, used = _run_tc(
        xs, tgt_slot.reshape(S, 1), rw8, W, tok_tile, w_blk, first, last,
        skip)
    nll, cls = _sc_unsort(nll_s.reshape(S), cls_s.reshape(S), slot)
    return (loss.reshape(()), used.reshape(()),
            nll.reshape(in_shape), cls.reshape(in_shape))
